# baseline (speedup 1.0000x reference)
import functools

import jax
import jax.numpy as jnp
from jax import lax
from jax.experimental import pallas as pl
from jax.experimental.pallas import tpu as pltpu
from jax.experimental.pallas import tpu_sc as plsc

MULT = 8.0
CHUNK = 128
K = 4
GROUP = K * CHUNK


def _make_sc_lookup(V, D, B):
    info = plsc.get_sparse_core_info()
    NC, NS, L = info.num_cores, info.num_subcores, info.num_lanes
    NW = NC * NS
    assert D % L == 0 and B % (NW * GROUP) == 0
    rows_per_w = B // NW
    groups = rows_per_w // GROUP
    chunks_per_w = rows_per_w // CHUNK
    mesh = plsc.VectorSubcoreMesh(core_axis_name="c", subcore_axis_name="s")

    @functools.partial(
        pl.kernel,
        mesh=mesh,
        out_type=jax.ShapeDtypeStruct((B, D), jnp.float32),
        compiler_params=pltpu.CompilerParams(use_tc_tiling_on_sc=False),
        scratch_types=[
            pltpu.VMEM((K, CHUNK), jnp.int32),
            pltpu.VMEM((GROUP, D), jnp.float32),
            pltpu.SemaphoreType.DMA,
        ],
    )
    def k(table_hbm, idx_hbm, out_hbm, idx_v, rows_v, sem):
        wid = lax.axis_index("s") * NC + lax.axis_index("c")

        def group_body(g, carry):
            pltpu.sync_copy(idx_hbm.at[wid, pl.ds(g * K, K)], idx_v)
            copies = [
                pltpu.async_copy(
                    table_hbm.at[idx_v.at[j]],
                    rows_v.at[pl.ds(j * CHUNK, CHUNK)],
                    sem,
                )
                for j in range(K)
            ]
            for c in copies:
                c.wait()

            def scale_body(r, carry2):
                for c4 in range(D // L):
                    sl = pl.ds(c4 * L, L)
                    rows_v[r, sl] = rows_v[r, sl] * MULT
                return carry2

            lax.fori_loop(0, GROUP, scale_body, 0)
            row_base = wid * rows_per_w + g * GROUP
            pltpu.sync_copy(rows_v, out_hbm.at[pl.ds(row_base, GROUP)])
            return carry

        lax.fori_loop(0, groups, group_body, 0)

    def run(weight, idx_flat):
        idx3 = idx_flat.reshape(NW, chunks_per_w, CHUNK)
        return k(weight, idx3)

    return run


def kernel(input_ids, weight):
    S0, S1 = input_ids.shape
    V, D = weight.shape
    B = S0 * S1
    idx_flat = input_ids.reshape(B).astype(jnp.int32)
    lookup = _make_sc_lookup(V, D, B)
    out = lookup(weight, idx_flat)
    return out.reshape(S0, S1, D)


# speedup vs baseline: 1.1374x; 1.1374x over previous
import functools

import jax
import jax.numpy as jnp
from jax import lax
from jax.experimental import pallas as pl
from jax.experimental.pallas import tpu as pltpu
from jax.experimental.pallas import tpu_sc as plsc

MULT = 8.0
CHUNK = 128
NBUF = 5


def _make_sc_lookup(V, D, B):
    info = plsc.get_sparse_core_info()
    NC, NS, L = info.num_cores, info.num_subcores, info.num_lanes
    NW = NC * NS
    assert D % L == 0 and B % (NW * CHUNK) == 0
    rows_per_w = B // NW
    chunks = rows_per_w // CHUNK
    assert chunks % NBUF == 0
    blocks = chunks // NBUF
    mesh = plsc.VectorSubcoreMesh(core_axis_name="c", subcore_axis_name="s")

    @functools.partial(
        pl.kernel,
        mesh=mesh,
        out_type=jax.ShapeDtypeStruct((B, D), jnp.float32),
        compiler_params=pltpu.CompilerParams(use_tc_tiling_on_sc=False),
        scratch_types=[
            pltpu.VMEM((chunks, CHUNK), jnp.int32),
            pltpu.VMEM((NBUF, CHUNK, D), jnp.float32),
            pltpu.VMEM((NBUF, CHUNK, D), jnp.float32),
            pltpu.SemaphoreType.DMA,
            pltpu.SemaphoreType.DMA,
        ],
    )
    def k(table_hbm, idx_hbm, out_hbm, idx_all, rows_g, rows_w, gsem, wsem):
        wid = lax.axis_index("s") * NC + lax.axis_index("c")
        row0 = wid * rows_per_w

        pltpu.sync_copy(idx_hbm.at[wid], idx_all)

        for b in range(NBUF):
            pltpu.async_copy(table_hbm.at[idx_all.at[b]], rows_g.at[b], gsem)

        def block_body(blk, carry):
            for b in range(NBUF):
                c = blk * NBUF + b
                pltpu.make_async_copy(
                    table_hbm.at[idx_all.at[0]], rows_g.at[b], gsem
                ).wait()

                @pl.when(blk > 0)
                def _wait_wb():
                    pltpu.make_async_copy(
                        rows_w.at[b], out_hbm.at[pl.ds(0, CHUNK)], wsem
                    ).wait()

                def scale_body(r, carry2):
                    for c4 in range(D // L):
                        sl = pl.ds(c4 * L, L)
                        rows_w[b, r, sl] = rows_g[b, r, sl] * MULT
                    return carry2

                lax.fori_loop(0, CHUNK, scale_body, 0)

                pltpu.async_copy(
                    rows_w.at[b], out_hbm.at[pl.ds(row0 + c * CHUNK, CHUNK)], wsem
                )

                @pl.when(c + NBUF < chunks)
                def _refill():
                    pltpu.async_copy(
                        table_hbm.at[idx_all.at[c + NBUF]], rows_g.at[b], gsem
                    )

            return carry

        lax.fori_loop(0, blocks, block_body, 0)

        for b in range(NBUF):
            pltpu.make_async_copy(
                rows_w.at[b], out_hbm.at[pl.ds(0, CHUNK)], wsem
            ).wait()

    def run(weight, idx_flat):
        idx3 = idx_flat.reshape(NW, chunks, CHUNK)
        return k(weight, idx3)

    return run


def kernel(input_ids, weight):
    S0, S1 = input_ids.shape
    V, D = weight.shape
    B = S0 * S1
    idx_flat = input_ids.reshape(B).astype(jnp.int32)
    lookup = _make_sc_lookup(V, D, B)
    out = lookup(weight, idx_flat)
    return out.reshape(S0, S1, D)


# speedup vs baseline: 1.4136x; 1.2428x over previous
import functools

import jax
import jax.numpy as jnp
from jax import lax
from jax.experimental import pallas as pl
from jax.experimental.pallas import tpu as pltpu
from jax.experimental.pallas import tpu_sc as plsc
from jax.experimental.layout import Layout, with_layout_constraint

MULT = 8.0
CHUNK = 128
NBUF = 5


def _make_sc_lookup(V, D, B):
    info = plsc.get_sparse_core_info()
    NC, NS, L = info.num_cores, info.num_subcores, info.num_lanes
    NW = NC * NS
    assert D % L == 0 and B % (NW * CHUNK) == 0
    rows_per_w = B // NW
    chunks = rows_per_w // CHUNK
    assert chunks % NBUF == 0
    mesh = plsc.VectorSubcoreMesh(core_axis_name="c", subcore_axis_name="s")

    @functools.partial(
        pl.kernel,
        mesh=mesh,
        out_type=jax.ShapeDtypeStruct((B, D), jnp.float32),
        compiler_params=pltpu.CompilerParams(use_tc_tiling_on_sc=False),
        scratch_types=[
            pltpu.VMEM((chunks, CHUNK), jnp.int32),
            pltpu.VMEM((NBUF, CHUNK, D), jnp.float32),
            pltpu.VMEM((NBUF, CHUNK, D), jnp.float32),
            pltpu.SemaphoreType.DMA,
            pltpu.SemaphoreType.DMA,
        ],
    )
    def k(table_hbm, idx_hbm, out_hbm, idx_all, rows_g, rows_w, gsem, wsem):
        wid = lax.axis_index("s") * NC + lax.axis_index("c")
        row0 = wid * rows_per_w

        pltpu.sync_copy(idx_hbm.at[wid], idx_all)

        for b in range(NBUF):
            pltpu.async_copy(table_hbm.at[idx_all.at[b]], rows_g.at[b], gsem)

        def block_body(blk, carry):
            for b in range(NBUF):
                c = blk * NBUF + b
                pltpu.make_async_copy(
                    table_hbm.at[idx_all.at[0]], rows_g.at[b], gsem
                ).wait()

                @pl.when(blk > 0)
                def _wait_wb():
                    pltpu.make_async_copy(
                        rows_w.at[b], out_hbm.at[pl.ds(0, CHUNK)], wsem
                    ).wait()

                def scale_body(r, carry2):
                    for c4 in range(D // L):
                        sl = pl.ds(c4 * L, L)
                        rows_w[b, r, sl] = rows_g[b, r, sl] * MULT
                    return carry2

                lax.fori_loop(0, CHUNK, scale_body, 0)

                pltpu.async_copy(
                    rows_w.at[b], out_hbm.at[pl.ds(row0 + c * CHUNK, CHUNK)], wsem
                )

                @pl.when(c + NBUF < chunks)
                def _refill():
                    pltpu.async_copy(
                        table_hbm.at[idx_all.at[c + NBUF]], rows_g.at[b], gsem
                    )

            return carry

        lax.fori_loop(0, chunks // NBUF, block_body, 0)

        for b in range(NBUF):
            pltpu.make_async_copy(
                rows_w.at[b], out_hbm.at[pl.ds(0, CHUNK)], wsem
            ).wait()

    def run(weight, idx_flat):
        w_rm = with_layout_constraint(weight, Layout((0, 1), tiling=((8,),)))
        idx3 = idx_flat.reshape(NW, chunks, CHUNK)
        return k(w_rm, idx3)

    return run


def kernel(input_ids, weight):
    S0, S1 = input_ids.shape
    V, D = weight.shape
    B = S0 * S1
    idx_flat = input_ids.reshape(B).astype(jnp.int32)
    lookup = _make_sc_lookup(V, D, B)
    out2d = lookup(weight, idx_flat)
    out = out2d.reshape(S0, S1, D)
    return with_layout_constraint(out, Layout((1, 2, 0), tiling=((8,),)))
